# baseline (speedup 1.0000x reference)
import functools

import jax
import jax.numpy as jnp
from jax import lax
from jax.experimental import pallas as pl
from jax.experimental.pallas import tpu as pltpu
from jax.experimental.pallas import tpu_sc as plsc

_L = 16
_IDX_CHUNK = 128


def _make_kernel(batch, factors, n_chunks):
    info = plsc.get_sparse_core_info()
    nw = info.num_cores * info.num_subcores
    b_per_w = batch // nw
    assert b_per_w % _IDX_CHUNK == 0 and b_per_w == n_chunks * _IDX_CHUNK
    idx_rows = b_per_w // _IDX_CHUNK

    mesh = plsc.VectorSubcoreMesh(core_axis_name="c", subcore_axis_name="s")

    @functools.partial(
        pl.kernel,
        mesh=mesh,
        out_type=jax.ShapeDtypeStruct((batch,), jnp.float32),
        compiler_params=pltpu.CompilerParams(
            needs_layout_passes=False, use_tc_tiling_on_sc=False),
        scratch_types=[
            pltpu.VMEM((idx_rows, _IDX_CHUNK), jnp.int32),
            pltpu.VMEM((idx_rows, _IDX_CHUNK), jnp.int32),
            pltpu.VMEM((b_per_w, factors), jnp.float32),
            pltpu.VMEM((b_per_w, factors), jnp.float32),
            pltpu.VMEM((b_per_w,), jnp.float32),
            pltpu.SemaphoreType.DMA,
        ],
    )
    def k(u_hbm, i_hbm, user_hbm, item_hbm, out_hbm,
          uidx_v, iidx_v, urows_v, irows_v, out_v, sem):
        wid = lax.axis_index("s") * info.num_cores + lax.axis_index("c")
        base = wid * b_per_w

        pltpu.sync_copy(u_hbm.at[pl.ds(wid * idx_rows, idx_rows)], uidx_v)
        pltpu.sync_copy(i_hbm.at[pl.ds(wid * idx_rows, idx_rows)], iidx_v)

        copies = []
        for j in range(idx_rows):
            copies.append(pltpu.async_copy(
                user_hbm.at[uidx_v.at[j]],
                urows_v.at[pl.ds(j * _IDX_CHUNK, _IDX_CHUNK)], sem))
            copies.append(pltpu.async_copy(
                item_hbm.at[iidx_v.at[j]],
                irows_v.at[pl.ds(j * _IDX_CHUNK, _IDX_CHUNK)], sem))
        for cp in copies:
            cp.wait()

        lane = lax.iota(jnp.int32, _L)

        for c in range(b_per_w // _L):
            rows = c * _L + lane
            acc = jnp.zeros((_L,), jnp.float32)
            for d in range(factors):
                col = jnp.full((_L,), d, jnp.int32)
                uc = plsc.load_gather(urows_v, [rows, col])
                ic = plsc.load_gather(irows_v, [rows, col])
                acc = acc + uc * ic
            out_v[pl.ds(c * _L, _L)] = acc

        pltpu.sync_copy(out_v, out_hbm.at[pl.ds(base, b_per_w)])

    return k


def kernel(u, i, user_table, item_table):
    batch = u.shape[0]
    factors = user_table.shape[1]
    info = plsc.get_sparse_core_info()
    nw = info.num_cores * info.num_subcores
    idx_rows = (batch // nw) // _IDX_CHUNK
    k = _make_kernel(batch, factors, idx_rows)
    u2 = u.reshape(nw * idx_rows, _IDX_CHUNK)
    i2 = i.reshape(nw * idx_rows, _IDX_CHUNK)
    rating = k(u2, i2, user_table, item_table)
    return rating.reshape(batch, 1, 1)
